# speedup vs baseline: 1.1439x; 1.1439x over previous
import jax
import jax.numpy as jnp
from jax.experimental import pallas as pl
from jax.experimental.pallas import tpu as pltpu

_NF = 256
_NL = 100000
_NC = 5000
_S = 10.0
_B = 256
_IGN = 5555
_T = 2048
_NLT = (_NL + _T - 1) // _T
_GRID = _NLT + 1


def _body(x_ref, tgt_ref, lut_ref, cq_ref,
          out_l_ref, out_c_ref, loss_ref, s_ref, g_ref):
    i = pl.program_id(0)

    @pl.when(i == 0)
    def _init():
        s_ref[...] = jnp.zeros_like(s_ref)
        g_ref[...] = jnp.zeros_like(g_ref)

    x = x_ref[...]
    tgt = tgt_ref[...]
    tgtc = jnp.clip(tgt, 0, _NL + _NC - 1)

    @pl.when(i < _NLT)
    def _lut_step():
        w = lut_ref[...]
        t = jax.lax.dot_general(
            x, w, (((1,), (1,)), ((), ())),
            preferred_element_type=jnp.float32) * _S
        out_l_ref[...] = t
        cols = i * _T + jax.lax.broadcasted_iota(jnp.int32, (_B, _T), 1)
        valid = cols < _NL
        e = jnp.where(valid, jnp.exp(t - _S), 0.0)
        s_ref[...] += jnp.sum(e, axis=1, keepdims=True)
        hit = valid & (cols == tgtc)
        g_ref[...] += jnp.sum(jnp.where(hit, t, 0.0), axis=1, keepdims=True)

    @pl.when(i == _NLT)
    def _cq_step():
        w = cq_ref[...]
        t = jax.lax.dot_general(
            x, w, (((1,), (1,)), ((), ())),
            preferred_element_type=jnp.float32) * _S
        out_c_ref[...] = t
        cols = _NL + jax.lax.broadcasted_iota(jnp.int32, (_B, _NC), 1)
        s = s_ref[...] + jnp.sum(jnp.exp(t - _S), axis=1, keepdims=True)
        hit = cols == tgtc
        g = g_ref[...] + jnp.sum(jnp.where(hit, t, 0.0), axis=1, keepdims=True)
        lse = _S + jnp.log(s)
        nll = lse - g
        w_cls = (tgtc < _NL).astype(jnp.float32)
        vmask = (tgt != _IGN).astype(jnp.float32)
        wgt = w_cls * vmask
        num = jnp.sum(nll * wgt)
        den = jnp.maximum(jnp.sum(wgt), 1.0)
        loss_ref[0, 0] = num / den


def kernel(inputs, targets, lut, cq):
    tgt2d = targets.reshape(_B, 1)
    out_l, out_c, loss = pl.pallas_call(
        _body,
        grid=(_GRID,),
        in_specs=[
            pl.BlockSpec((_B, _NF), lambda i: (0, 0)),
            pl.BlockSpec((_B, 1), lambda i: (0, 0)),
            pl.BlockSpec((_T, _NF), lambda i: (jnp.minimum(i, _NLT - 1), 0)),
            pl.BlockSpec((_NC, _NF), lambda i: (0, 0)),
        ],
        out_specs=[
            pl.BlockSpec((_B, _T), lambda i: (0, jnp.minimum(i, _NLT - 1))),
            pl.BlockSpec((_B, _NC), lambda i: (0, 0)),
            pl.BlockSpec(memory_space=pltpu.SMEM),
        ],
        out_shape=[
            jax.ShapeDtypeStruct((_B, _NL + _NC), jnp.float32),
            jax.ShapeDtypeStruct((_B, _NC), jnp.float32),
            jax.ShapeDtypeStruct((1, 1), jnp.float32),
        ],
        scratch_shapes=[
            pltpu.VMEM((_B, 1), jnp.float32),
            pltpu.VMEM((_B, 1), jnp.float32),
        ],
        compiler_params=pltpu.CompilerParams(
            dimension_semantics=("arbitrary",),
        ),
    )(inputs, tgt2d, lut, cq)
    logits = jax.lax.dynamic_update_slice(out_l, out_c, (0, _NL))
    return loss[0, 0], logits


# speedup vs baseline: 1.2356x; 1.0801x over previous
import jax
import jax.numpy as jnp
from jax.experimental import pallas as pl
from jax.experimental.pallas import tpu as pltpu

_NF = 256
_NL = 100000
_NC = 5000
_NTOT = _NL + _NC
_S = 10.0
_B = 256
_IGN = 5555
_T = 2048
_NFULL = _NL // _T
_TAIL0 = _NFULL * _T
_NTAIL = (_NTOT - _TAIL0 + _T - 1) // _T
_GRID = _NFULL + _NTAIL


def _body(x_ref, tgt_ref, lut_ref, tail_ref,
          out_ref, loss_ref, s_ref, g_ref):
    i = pl.program_id(0)

    @pl.when(i == 0)
    def _init():
        s_ref[...] = jnp.zeros_like(s_ref)
        g_ref[...] = jnp.zeros_like(g_ref)

    x = x_ref[...]
    tgt = tgt_ref[...]
    tgtc = jnp.clip(tgt, 0, _NTOT - 1)

    def _step(w):
        t = jax.lax.dot_general(
            x, w, (((1,), (1,)), ((), ())),
            preferred_element_type=jnp.float32) * _S
        out_ref[...] = t
        cols = i * _T + jax.lax.broadcasted_iota(jnp.int32, (_B, _T), 1)
        valid = cols < _NTOT
        e = jnp.where(valid, jnp.exp(t - _S), 0.0)
        s_ref[...] += jnp.sum(e, axis=1, keepdims=True)
        hit = valid & (cols == tgtc)
        g_ref[...] += jnp.sum(jnp.where(hit, t, 0.0), axis=1, keepdims=True)

    @pl.when(i < _NFULL)
    def _lut_step():
        _step(lut_ref[...])

    @pl.when(i >= _NFULL)
    def _tail_step():
        _step(tail_ref[...])

    @pl.when(i == _GRID - 1)
    def _finalize():
        lse = _S + jnp.log(s_ref[...])
        nll = lse - g_ref[...]
        w_cls = (tgtc < _NL).astype(jnp.float32)
        vmask = (tgt != _IGN).astype(jnp.float32)
        wgt = w_cls * vmask
        num = jnp.sum(nll * wgt)
        den = jnp.maximum(jnp.sum(wgt), 1.0)
        loss_ref[0, 0] = num / den


def kernel(inputs, targets, lut, cq):
    tgt2d = targets.reshape(_B, 1)
    tail = jnp.concatenate([lut[_TAIL0:], cq], axis=0)
    out, loss = pl.pallas_call(
        _body,
        grid=(_GRID,),
        in_specs=[
            pl.BlockSpec((_B, _NF), lambda i: (0, 0)),
            pl.BlockSpec((_B, 1), lambda i: (0, 0)),
            pl.BlockSpec((_T, _NF), lambda i: (jnp.minimum(i, _NFULL - 1), 0)),
            pl.BlockSpec((_T, _NF),
                         lambda i: (jnp.clip(i - _NFULL, 0, _NTAIL - 1), 0)),
        ],
        out_specs=[
            pl.BlockSpec((_B, _T), lambda i: (0, i)),
            pl.BlockSpec(memory_space=pltpu.SMEM),
        ],
        out_shape=[
            jax.ShapeDtypeStruct((_B, _NTOT), jnp.float32),
            jax.ShapeDtypeStruct((1, 1), jnp.float32),
        ],
        scratch_shapes=[
            pltpu.VMEM((_B, 1), jnp.float32),
            pltpu.VMEM((_B, 1), jnp.float32),
        ],
        compiler_params=pltpu.CompilerParams(
            dimension_semantics=("arbitrary",),
        ),
    )(inputs, tgt2d, lut, tail)
    return loss[0, 0], out


# speedup vs baseline: 1.2480x; 1.0100x over previous
import jax
import jax.numpy as jnp
from jax.experimental import pallas as pl
from jax.experimental.pallas import tpu as pltpu

_NF = 256
_NL = 100000
_NC = 5000
_NTOT = _NL + _NC
_S = 10.0
_B = 256
_IGN = 5555
_T = 2048
_NFULL = _NL // _T
_TAIL0 = _NFULL * _T
_NTAIL = (_NTOT - _TAIL0 + _T - 1) // _T
_GRID = _NFULL + _NTAIL


def _body(x_ref, tgt_ref, lut_ref, tail_ref,
          out_ref, loss_ref, s_ref, g_ref):
    i = pl.program_id(0)

    @pl.when(i == 0)
    def _init():
        s_ref[...] = jnp.zeros_like(s_ref)
        g_ref[...] = jnp.zeros_like(g_ref)

    x = x_ref[...]
    tgt = tgt_ref[...]
    tgtc = jnp.clip(tgt, 0, _NTOT - 1)

    def _step(w, mask_tail):
        t = jax.lax.dot_general(
            x, w, (((1,), (1,)), ((), ())),
            preferred_element_type=jnp.float32) * _S
        out_ref[...] = t
        cols = i * _T + jax.lax.broadcasted_iota(jnp.int32, (_B, _T), 1)
        if mask_tail:
            e = jnp.where(cols < _NTOT, jnp.exp(t - _S), 0.0)
        else:
            e = jnp.exp(t - _S)
        s_ref[...] += jnp.sum(e, axis=1, keepdims=True)
        hit = cols == tgtc
        g_ref[...] += jnp.sum(jnp.where(hit, t, 0.0), axis=1, keepdims=True)

    @pl.when(i < _NFULL)
    def _lut_step():
        _step(lut_ref[...], False)

    @pl.when(jnp.logical_and(i >= _NFULL, i < _GRID - 1))
    def _tail_step():
        _step(tail_ref[...], False)

    @pl.when(i == _GRID - 1)
    def _last_step():
        _step(tail_ref[...], True)

    @pl.when(i == _GRID - 1)
    def _finalize():
        lse = _S + jnp.log(s_ref[...])
        nll = lse - g_ref[...]
        w_cls = (tgtc < _NL).astype(jnp.float32)
        vmask = (tgt != _IGN).astype(jnp.float32)
        wgt = w_cls * vmask
        num = jnp.sum(nll * wgt)
        den = jnp.maximum(jnp.sum(wgt), 1.0)
        loss_ref[0, 0] = num / den


def kernel(inputs, targets, lut, cq):
    tgt2d = targets.reshape(_B, 1)
    tail = jnp.concatenate([lut[_TAIL0:], cq], axis=0)
    out, loss = pl.pallas_call(
        _body,
        grid=(_GRID,),
        in_specs=[
            pl.BlockSpec((_B, _NF), lambda i: (0, 0)),
            pl.BlockSpec((_B, 1), lambda i: (0, 0)),
            pl.BlockSpec((_T, _NF), lambda i: (jnp.minimum(i, _NFULL - 1), 0)),
            pl.BlockSpec((_T, _NF),
                         lambda i: (jnp.clip(i - _NFULL, 0, _NTAIL - 1), 0)),
        ],
        out_specs=[
            pl.BlockSpec((_B, _T), lambda i: (0, i)),
            pl.BlockSpec(memory_space=pltpu.SMEM),
        ],
        out_shape=[
            jax.ShapeDtypeStruct((_B, _NTOT), jnp.float32),
            jax.ShapeDtypeStruct((1, 1), jnp.float32),
        ],
        scratch_shapes=[
            pltpu.VMEM((_B, 1), jnp.float32),
            pltpu.VMEM((_B, 1), jnp.float32),
        ],
        compiler_params=pltpu.CompilerParams(
            dimension_semantics=("arbitrary",),
        ),
    )(inputs, tgt2d, lut, tail)
    return loss[0, 0], out
